# baseline (speedup 1.0000x reference)
import functools

import jax
import jax.numpy as jnp
from jax import lax
from jax.experimental import pallas as pl
from jax.experimental.pallas import tpu as pltpu
from jax.experimental.pallas import tpu_sc as plsc

VOCAB = 1000000
MAXLEN = 200
D = 64
BATCH = 4096
SEQ = 200

NC, NS = 2, 16
NW = NC * NS
ROWS = BATCH * SEQ
ROWS_PER_W = ROWS // NW
CH = SEQ
N_CH = ROWS_PER_W // CH
LANES = 16


def _body(x_hbm, tok_hbm, pos_hbm, out_hbm, idx_v, rows_v, pos_v, sem):
    wid = lax.axis_index("s") * NC + lax.axis_index("c")
    base = wid * ROWS_PER_W

    pltpu.sync_copy(pos_hbm, pos_v)

    def chunk_body(i, carry):
        off = base + i * CH
        pltpu.sync_copy(x_hbm.at[pl.ds(off, CH)], idx_v)
        pltpu.async_copy(tok_hbm.at[idx_v], rows_v, sem).wait()

        def add_row(r, c):
            for j in range(D // LANES):
                sl = pl.ds(j * LANES, LANES)
                rows_v[r, sl] = rows_v[r, sl] + pos_v[r, sl]
            return c

        lax.fori_loop(0, CH, add_row, 0)
        pltpu.sync_copy(rows_v, out_hbm.at[pl.ds(off, CH)])
        return carry

    lax.fori_loop(0, N_CH, chunk_body, 0)


@jax.jit
def _embed(x_flat, token_emb, pos_emb):
    mesh = plsc.VectorSubcoreMesh(core_axis_name="c", subcore_axis_name="s")
    f = pl.kernel(
        _body,
        out_type=jax.ShapeDtypeStruct((ROWS, D), jnp.float32),
        mesh=mesh,
        scratch_types=[
            pltpu.VMEM((CH,), jnp.int32),
            pltpu.VMEM((CH, D), jnp.float32),
            pltpu.VMEM((SEQ, D), jnp.float32),
            pltpu.SemaphoreType.DMA,
        ],
        compiler_params=pltpu.CompilerParams(use_tc_tiling_on_sc=False),
    )
    return f(x_flat, token_emb, pos_emb)


def kernel(x, token_emb, pos_emb):
    x_flat = x.reshape(ROWS).astype(jnp.int32)
    out = _embed(x_flat, token_emb, pos_emb)
    return out.reshape(BATCH, SEQ, D)


# speedup vs baseline: 1.1576x; 1.1576x over previous
import jax
import jax.numpy as jnp
from jax import lax
from jax.experimental import pallas as pl
from jax.experimental.pallas import tpu as pltpu
from jax.experimental.pallas import tpu_sc as plsc

VOCAB = 1000000
MAXLEN = 200
D = 64
BATCH = 4096
SEQ = 200

NC, NS = 2, 16
NW = NC * NS
ROWS = BATCH * SEQ
ROWS_PER_W = ROWS // NW
CH = SEQ
N_CH = ROWS_PER_W // CH
NBUF = 4
N_WAVES = N_CH // NBUF
LANES = 16


def _body(x_hbm, tok_hbm, pos_hbm, out_hbm, idx_v, rows_v, pos_v,
          sem_g, sem_s):
    wid = lax.axis_index("s") * NC + lax.axis_index("c")
    base = wid * ROWS_PER_W

    pltpu.sync_copy(pos_hbm, pos_v)

    def wave(g, carry):
        for b in range(NBUF):
            off = base + (g * NBUF + b) * CH

            @pl.when(g > 0)
            def _wait_scatter(b=b, off=off):
                pltpu.make_async_copy(
                    rows_v.at[b], out_hbm.at[pl.ds(off, CH)], sem_s[b]
                ).wait()

            pltpu.sync_copy(x_hbm.at[pl.ds(off, CH)], idx_v.at[b])
            pltpu.async_copy(tok_hbm.at[idx_v.at[b]], rows_v.at[b], sem_g[b])

        for b in range(NBUF):
            off = base + (g * NBUF + b) * CH
            pltpu.make_async_copy(
                tok_hbm.at[idx_v.at[b]], rows_v.at[b], sem_g[b]
            ).wait()

            @plsc.parallel_loop(0, CH, step=1, unroll=4)
            def _add_row(r, b=b):
                for j in range(D // LANES):
                    sl = pl.ds(j * LANES, LANES)
                    rows_v[b, r, sl] = rows_v[b, r, sl] + pos_v[r, sl]

            pltpu.async_copy(rows_v.at[b], out_hbm.at[pl.ds(off, CH)],
                             sem_s[b])
        return carry

    lax.fori_loop(0, N_WAVES, wave, 0)

    for b in range(NBUF):
        pltpu.make_async_copy(
            rows_v.at[b], out_hbm.at[pl.ds(base, CH)], sem_s[b]
        ).wait()


@jax.jit
def _embed(x_flat, token_emb, pos_emb):
    mesh = plsc.VectorSubcoreMesh(core_axis_name="c", subcore_axis_name="s")
    f = pl.kernel(
        _body,
        out_type=jax.ShapeDtypeStruct((ROWS, D), jnp.float32),
        mesh=mesh,
        scratch_types=[
            pltpu.VMEM((NBUF, CH), jnp.int32),
            pltpu.VMEM((NBUF, CH, D), jnp.float32),
            pltpu.VMEM((SEQ, D), jnp.float32),
            [pltpu.SemaphoreType.DMA] * NBUF,
            [pltpu.SemaphoreType.DMA] * NBUF,
        ],
        compiler_params=pltpu.CompilerParams(use_tc_tiling_on_sc=False),
    )
    return f(x_flat, token_emb, pos_emb)


def kernel(x, token_emb, pos_emb):
    x_flat = x.reshape(ROWS).astype(jnp.int32)
    out = _embed(x_flat, token_emb, pos_emb)
    return out.reshape(BATCH, SEQ, D)


# speedup vs baseline: 1.1602x; 1.0022x over previous
import jax
import jax.numpy as jnp
from jax import lax
from jax.experimental import pallas as pl
from jax.experimental.pallas import tpu as pltpu
from jax.experimental.pallas import tpu_sc as plsc

VOCAB = 1000000
MAXLEN = 200
D = 64
BATCH = 4096
SEQ = 200

NC, NS = 2, 16
NW = NC * NS
ROWS = BATCH * SEQ
ROWS_PER_W = ROWS // NW
CH = SEQ
N_CH = ROWS_PER_W // CH
NBUF = 4
N_WAVES = N_CH // NBUF
LANES = 16


def _body(x_hbm, tok_hbm, pos_hbm, out_hbm, idx_v, rows_v, pos_v,
          sem_g, sem_s):
    wid = lax.axis_index("s") * NC + lax.axis_index("c")
    base = wid * ROWS_PER_W

    pltpu.sync_copy(x_hbm.at[pl.ds(base, ROWS_PER_W)], idx_v)
    pltpu.sync_copy(pos_hbm, pos_v)

    def wave(g, carry):
        for b in range(NBUF):
            i = g * NBUF + b

            @pl.when(g > 0)
            def _wait_scatter(b=b, i=i):
                pltpu.make_async_copy(
                    rows_v.at[b], out_hbm.at[pl.ds(base + i * CH, CH)],
                    sem_s[b]
                ).wait()

            pltpu.async_copy(tok_hbm.at[idx_v.at[pl.ds(i * CH, CH)]],
                             rows_v.at[b], sem_g[b])

        for b in range(NBUF):
            i = g * NBUF + b
            pltpu.make_async_copy(
                tok_hbm.at[idx_v.at[pl.ds(i * CH, CH)]], rows_v.at[b],
                sem_g[b]
            ).wait()

            @plsc.parallel_loop(0, CH, step=1, unroll=4)
            def _add_row(r, b=b):
                for j in range(D // LANES):
                    sl = pl.ds(j * LANES, LANES)
                    rows_v[b, r, sl] = rows_v[b, r, sl] + pos_v[r, sl]

            pltpu.async_copy(rows_v.at[b],
                             out_hbm.at[pl.ds(base + i * CH, CH)], sem_s[b])
        return carry

    lax.fori_loop(0, N_WAVES, wave, 0)

    for b in range(NBUF):
        pltpu.make_async_copy(
            rows_v.at[b], out_hbm.at[pl.ds(base, CH)], sem_s[b]
        ).wait()


@jax.jit
def _embed(x_flat, token_emb, pos_emb):
    mesh = plsc.VectorSubcoreMesh(core_axis_name="c", subcore_axis_name="s")
    f = pl.kernel(
        _body,
        out_type=jax.ShapeDtypeStruct((ROWS, D), jnp.float32),
        mesh=mesh,
        scratch_types=[
            pltpu.VMEM((ROWS_PER_W,), jnp.int32),
            pltpu.VMEM((NBUF, CH, D), jnp.float32),
            pltpu.VMEM((SEQ, D), jnp.float32),
            [pltpu.SemaphoreType.DMA] * NBUF,
            [pltpu.SemaphoreType.DMA] * NBUF,
        ],
        compiler_params=pltpu.CompilerParams(use_tc_tiling_on_sc=False),
    )
    return f(x_flat, token_emb, pos_emb)


def kernel(x, token_emb, pos_emb):
    x_flat = x.reshape(ROWS).astype(jnp.int32)
    out = _embed(x_flat, token_emb, pos_emb)
    return out.reshape(BATCH, SEQ, D)
